# Initial kernel scaffold; baseline (speedup 1.0000x reference)
import jax
import jax.numpy as jnp
from jax.experimental import pallas as pl


def kernel(in_feat, edge_index, W_ih1, W_hh1, b_ih1, b_hh1, W_ih2, W_hh2, b_ih2, b_hh2, W1, b1, W2, b2, W3, b3):
    raise NotImplementedError("write your pallas kernel here")


# speedup vs baseline: 4.7105x; 4.7105x over previous
import functools

import jax
import jax.numpy as jnp
from jax import lax
from jax.experimental import pallas as pl
from jax.experimental.pallas import tpu as pltpu
from jax.experimental.pallas import tpu_sc as plsc

N_NODES = 10000
N_EDGES = 160000
SEQ = 16
HID1 = 32
HID2 = 16

NC = 2
NS = 16
NW = NC * NS
CHUNK = 128
NCHUNK = 40
EPW = CHUNK * NCHUNK
E_PAD = EPW * NW
ROWS_PER_TILE = 632
N_PAD = ROWS_PER_TILE * NS


def _lstm_body(x_ref, wih1_ref, whh1_ref, b1_ref, wih2_ref, whh2_ref,
               b2_ref, w1_ref, out_ref):
    x = x_ref[...]
    wih1 = wih1_ref[...]
    whh1 = whh1_ref[...]
    b1 = b1_ref[...]
    wih2 = wih2_ref[...]
    whh2 = whh2_ref[...]
    b2 = b2_ref[...]
    B = x.shape[0]

    h = jnp.zeros((B, HID1), jnp.float32)
    c = jnp.zeros((B, HID1), jnp.float32)
    h1s = []
    for t in range(SEQ):
        gates = (x[:, t:t + 1] * wih1
                 + jnp.dot(h, whh1, preferred_element_type=jnp.float32) + b1)
        i = jax.nn.sigmoid(gates[:, 0:32])
        f = jax.nn.sigmoid(gates[:, 32:64])
        g = jnp.tanh(gates[:, 64:96])
        o = jax.nn.sigmoid(gates[:, 96:128])
        c = f * c + i * g
        h = o * jnp.tanh(c)
        h1s.append(h)

    h2 = jnp.zeros((B, HID2), jnp.float32)
    c2 = jnp.zeros((B, HID2), jnp.float32)
    acc = jnp.zeros((B, 16), jnp.float32)
    for t in range(SEQ):
        gates = (jnp.dot(h1s[t], wih2, preferred_element_type=jnp.float32)
                 + jnp.dot(h2, whh2, preferred_element_type=jnp.float32) + b2)
        i = jax.nn.sigmoid(gates[:, 0:16])
        f = jax.nn.sigmoid(gates[:, 16:32])
        g = jnp.tanh(gates[:, 32:48])
        o = jax.nn.sigmoid(gates[:, 48:64])
        c2 = f * c2 + i * g
        h2 = o * jnp.tanh(c2)
        acc = acc + jnp.dot(h2, w1_ref[t], preferred_element_type=jnp.float32)
    out_ref[...] = acc


def _lstm_project(in_feat, wih1, whh1, b1, wih2, whh2, b2, w1):
    BN = 2000
    grid = (N_NODES // BN,)
    full = lambda shape: pl.BlockSpec(shape, lambda i: (0,) * len(shape))
    return pl.pallas_call(
        _lstm_body,
        grid=grid,
        in_specs=[
            pl.BlockSpec((BN, SEQ), lambda i: (i, 0)),
            full((1, 128)), full((HID1, 128)), full((1, 128)),
            full((HID1, 64)), full((HID2, 64)), full((1, 64)),
            full((SEQ, 16, 16)),
        ],
        out_specs=pl.BlockSpec((BN, 16), lambda i: (i, 0)),
        out_shape=jax.ShapeDtypeStruct((N_NODES, 16), jnp.float32),
    )(in_feat, wih1, whh1, b1, wih2, whh2, b2, w1)


def _agg_body(with_deg, y_hbm, srcs_hbm, dsts_hbm, out_hbm, *rest):
    if with_deg:
        deg_hbm = rest[0]
        rest = rest[1:]
    src_v, dst_v, rows_v, stripe_v, ones_v, acc_sh, deg_sh, sem = rest

    c = lax.axis_index("c")
    s = lax.axis_index("s")
    wid = s * NC + c

    def _zrow(i, _):
        stripe_v[i, :] = jnp.zeros((16,), jnp.float32)
        return 0
    lax.fori_loop(0, ROWS_PER_TILE, _zrow, 0)
    pltpu.sync_copy(stripe_v, acc_sh.at[pl.ds(s * ROWS_PER_TILE, ROWS_PER_TILE)])
    if with_deg:
        pltpu.sync_copy(stripe_v,
                        deg_sh.at[pl.ds(s * ROWS_PER_TILE, ROWS_PER_TILE)])

        def _orow(i, _):
            ones_v[i, :] = jnp.ones((16,), jnp.float32)
            return 0
        lax.fori_loop(0, CHUNK, _orow, 0)

    pltpu.sync_copy(srcs_hbm.at[wid], src_v)
    pltpu.sync_copy(dsts_hbm.at[wid], dst_v)
    plsc.subcore_barrier()

    def _chunk(j, _):
        pltpu.async_copy(y_hbm.at[src_v.at[j]], rows_v, sem).wait()
        pltpu.sync_copy(rows_v, acc_sh.at[dst_v.at[j]], add=True)
        if with_deg:
            pltpu.sync_copy(ones_v, deg_sh.at[dst_v.at[j]], add=True)
        return 0
    lax.fori_loop(0, NCHUNK, _chunk, 0)
    plsc.subcore_barrier()

    sl = pl.ds(s * ROWS_PER_TILE, ROWS_PER_TILE)
    pltpu.sync_copy(acc_sh.at[sl], stripe_v)
    pltpu.sync_copy(stripe_v, out_hbm.at[c, sl])
    if with_deg:
        pltpu.sync_copy(deg_sh.at[sl], stripe_v)
        pltpu.sync_copy(stripe_v, deg_hbm.at[c, sl])


@functools.lru_cache(maxsize=None)
def _make_agg(with_deg):
    part = jax.ShapeDtypeStruct((NC, N_PAD, 16), jnp.float32)
    out_type = (part, part) if with_deg else part
    return pl.kernel(
        functools.partial(_agg_body, with_deg),
        out_type=out_type,
        mesh=plsc.VectorSubcoreMesh(core_axis_name="c", subcore_axis_name="s",
                                    num_cores=NC, num_subcores=NS),
        scratch_types=[
            pltpu.VMEM((NCHUNK, CHUNK), jnp.int32),
            pltpu.VMEM((NCHUNK, CHUNK), jnp.int32),
            pltpu.VMEM((CHUNK, 16), jnp.float32),
            pltpu.VMEM((ROWS_PER_TILE, 16), jnp.float32),
            pltpu.VMEM((CHUNK, 16), jnp.float32),
            pltpu.VMEM_SHARED((N_PAD, 16), jnp.float32),
            pltpu.VMEM_SHARED((N_PAD, 16), jnp.float32),
            pltpu.SemaphoreType.DMA,
        ],
        compiler_params=pltpu.CompilerParams(use_tc_tiling_on_sc=False),
    )


def _blockdiag(w):
    tiled = jnp.tile(w, (8, 8))
    r = lax.broadcasted_iota(jnp.int32, (128, 128), 0) // 16
    col = lax.broadcasted_iota(jnp.int32, (128, 128), 1) // 16
    return jnp.where(r == col, tiled, 0.0)


def _node_mid_body(y_ref, p0_ref, p1_ref, d0_ref, d1_ref, w_ref, b_ref, o_ref):
    h = ((p0_ref[...] + p1_ref[...] + y_ref[...])
         / (d0_ref[...] + d1_ref[...] + 1.0) + jnp.tile(b_ref[...], (1, 8)))
    o_ref[...] = jnp.dot(h, _blockdiag(w_ref[...]),
                         preferred_element_type=jnp.float32)


def _node_last_body(y_ref, p0_ref, p1_ref, d0_ref, d1_ref, b_ref, o_ref):
    h = ((p0_ref[...] + p1_ref[...] + y_ref[...])
         / (d0_ref[...] + d1_ref[...] + 1.0) + jnp.tile(b_ref[...], (1, 8)))
    o_ref[...] = jnp.maximum(h, 0.0)


def _node_mid(y, p0, p1, d0, d1, w, b):
    return pl.pallas_call(
        _node_mid_body,
        out_shape=jax.ShapeDtypeStruct((1250, 128), jnp.float32),
    )(y, p0, p1, d0, d1, w, b.reshape(1, 16))


def _node_last(y, p0, p1, d0, d1, b):
    return pl.pallas_call(
        _node_last_body,
        out_shape=jax.ShapeDtypeStruct((1250, 128), jnp.float32),
    )(y, p0, p1, d0, d1, b.reshape(1, 16))


def kernel(in_feat, edge_index, W_ih1, W_hh1, b_ih1, b_hh1,
           W_ih2, W_hh2, b_ih2, b_hh2, W1, b1, W2, b2, W3, b3):
    f32 = jnp.float32
    src = edge_index[0].astype(jnp.int32)
    dst = edge_index[1].astype(jnp.int32)
    pad = E_PAD - N_EDGES
    srcs = jnp.concatenate([src, jnp.zeros((pad,), jnp.int32)])
    dsts = jnp.concatenate([dst, jnp.full((pad,), N_NODES, jnp.int32)])
    srcs = srcs.reshape(NW, NCHUNK, CHUNK)
    dsts = dsts.reshape(NW, NCHUNK, CHUNK)

    y1 = _lstm_project(
        in_feat,
        W_ih1.T.reshape(1, 128),
        W_hh1.T,
        (b_ih1 + b_hh1).reshape(1, 128),
        W_ih2.T,
        W_hh2.T,
        (b_ih2 + b_hh2).reshape(1, 64),
        W1.reshape(SEQ, 16, 16),
    )

    a1, dp = _make_agg(True)(y1, srcs, dsts)
    v = lambda p: p[:, :N_NODES, :].reshape(NC, 1250, 128)
    d = v(dp)
    a1 = v(a1)
    y1r = y1.reshape(1250, 128)

    y2r = _node_mid(y1r, a1[0], a1[1], d[0], d[1], W2, b1)
    a2 = v(_make_agg(False)(y2r.reshape(N_NODES, 16), srcs, dsts))
    y3r = _node_mid(y2r, a2[0], a2[1], d[0], d[1], W3, b2)
    a3 = v(_make_agg(False)(y3r.reshape(N_NODES, 16), srcs, dsts))
    outr = _node_last(y3r, a3[0], a3[1], d[0], d[1], b3)
    return outr.reshape(N_NODES, 16).astype(f32)


# speedup vs baseline: 8.5014x; 1.8048x over previous
import functools

import jax
import jax.numpy as jnp
from jax import lax
from jax.experimental import pallas as pl
from jax.experimental.pallas import tpu as pltpu
from jax.experimental.pallas import tpu_sc as plsc

N_NODES = 10000
N_EDGES = 160000
SEQ = 16
HID1 = 32
HID2 = 16

NC = 2
NS = 16
NW = NC * NS
CHUNK = 128
NCHUNK = 40
EPW = CHUNK * NCHUNK
E_PAD = EPW * NW
ROWS_PER_TILE = 632
N_PAD = ROWS_PER_TILE * NS


def _lstm_body(x_ref, wih1_ref, whh1_ref, b1_ref, wih2_ref, whh2_ref,
               b2_ref, w1_ref, out_ref):
    x = x_ref[...]
    wih1 = wih1_ref[...]
    whh1 = whh1_ref[...]
    b1 = b1_ref[...]
    wih2 = wih2_ref[...]
    whh2 = whh2_ref[...]
    b2 = b2_ref[...]
    B = x.shape[1]

    h = jnp.zeros((HID1, B), jnp.float32)
    c = jnp.zeros((HID1, B), jnp.float32)
    h1s = []
    for t in range(SEQ):
        gates = (wih1 * x[t:t + 1, :]
                 + jnp.dot(whh1, h, preferred_element_type=jnp.float32) + b1)
        sio = jax.nn.sigmoid(gates[0:96, :])
        g = jnp.tanh(gates[96:128, :])
        c = sio[32:64, :] * c + sio[0:32, :] * g
        h = sio[64:96, :] * jnp.tanh(c)
        h1s.append(h)

    h2 = jnp.zeros((HID2, B), jnp.float32)
    c2 = jnp.zeros((HID2, B), jnp.float32)
    acc = jnp.zeros((16, B), jnp.float32)
    for t in range(SEQ):
        gates = (jnp.dot(wih2, h1s[t], preferred_element_type=jnp.float32)
                 + jnp.dot(whh2, h2, preferred_element_type=jnp.float32) + b2)
        sio = jax.nn.sigmoid(gates[0:48, :])
        g = jnp.tanh(gates[48:64, :])
        c2 = sio[16:32, :] * c2 + sio[0:16, :] * g
        h2 = sio[32:48, :] * jnp.tanh(c2)
        acc = acc + jnp.dot(w1_ref[t], h2, preferred_element_type=jnp.float32)
    out_ref[...] = acc


N_LANE_PAD = 10240


def _lstm_project(xT, wih1, whh1, b1, wih2, whh2, b2, w1):
    BN = 2048
    grid = (N_LANE_PAD // BN,)
    full = lambda shape: pl.BlockSpec(shape, lambda i: (0,) * len(shape))
    return pl.pallas_call(
        _lstm_body,
        grid=grid,
        in_specs=[
            pl.BlockSpec((SEQ, BN), lambda i: (0, i)),
            full((128, 1)), full((128, HID1)), full((128, 1)),
            full((64, HID1)), full((64, HID2)), full((64, 1)),
            full((SEQ, 16, 16)),
        ],
        out_specs=pl.BlockSpec((16, BN), lambda i: (0, i)),
        out_shape=jax.ShapeDtypeStruct((16, N_LANE_PAD), jnp.float32),
    )(xT, wih1, whh1, b1, wih2, whh2, b2, w1)


def _perm_gates(w, n):
    return w.reshape(4, n, *w.shape[1:])[jnp.array([0, 1, 3, 2])].reshape(w.shape)


NBUF = 4


def _agg_body(with_deg, y_hbm, srcs_hbm, dsts_hbm, out_hbm, *rest):
    if with_deg:
        deg_hbm = rest[0]
        rest = rest[1:]
    src_v, dst_v, r0, r1, r2, r3, stripe_v, ones_v, acc_sh, deg_sh = rest[:10]
    sems = rest[10:]
    gsem = sems[:NBUF]
    ssem = sems[NBUF:]
    rows = [r0, r1, r2, r3]

    c = lax.axis_index("c")
    s = lax.axis_index("s")
    wid = s * NC + c

    def _zrow(i, _):
        stripe_v[i, :] = jnp.zeros((16,), jnp.float32)
        return 0
    lax.fori_loop(0, ROWS_PER_TILE, _zrow, 0)
    pltpu.sync_copy(stripe_v, acc_sh.at[pl.ds(s * ROWS_PER_TILE, ROWS_PER_TILE)])
    if with_deg:
        pltpu.sync_copy(stripe_v,
                        deg_sh.at[pl.ds(s * ROWS_PER_TILE, ROWS_PER_TILE)])

        def _orow(i, _):
            ones_v[i, :] = jnp.ones((16,), jnp.float32)
            return 0
        lax.fori_loop(0, CHUNK, _orow, 0)

    pltpu.sync_copy(srcs_hbm.at[wid], src_v)
    pltpu.sync_copy(dsts_hbm.at[wid], dst_v)
    plsc.subcore_barrier()

    for b in range(NBUF):
        pltpu.async_copy(y_hbm.at[src_v.at[b]], rows[b], gsem[b])

    def _round(k, _):
        base = k * NBUF
        for b in range(NBUF):
            j = base + b
            pltpu.make_async_copy(y_hbm.at[src_v.at[j]], rows[b],
                                  gsem[b]).wait()
            pltpu.async_copy(rows[b], acc_sh.at[dst_v.at[j]], ssem[b],
                             add=True)
            if with_deg:
                pltpu.async_copy(ones_v, deg_sh.at[dst_v.at[j]], ssem[b],
                                 add=True)

        @pl.when(k < NCHUNK // NBUF - 1)
        def _refill():
            for b in range(NBUF):
                j = base + b
                pltpu.make_async_copy(rows[b], acc_sh.at[dst_v.at[j]],
                                      ssem[b]).wait()
                if with_deg:
                    pltpu.make_async_copy(ones_v, deg_sh.at[dst_v.at[j]],
                                          ssem[b]).wait()
                pltpu.async_copy(y_hbm.at[src_v.at[j + NBUF]], rows[b],
                                 gsem[b])
        return 0
    lax.fori_loop(0, NCHUNK // NBUF, _round, 0)

    for b in range(NBUF):
        j = NCHUNK - NBUF + b
        pltpu.make_async_copy(rows[b], acc_sh.at[dst_v.at[j]], ssem[b]).wait()
        if with_deg:
            pltpu.make_async_copy(ones_v, deg_sh.at[dst_v.at[j]],
                                  ssem[b]).wait()
    plsc.subcore_barrier()

    sl = pl.ds(s * ROWS_PER_TILE, ROWS_PER_TILE)
    pltpu.sync_copy(acc_sh.at[sl], stripe_v)
    pltpu.sync_copy(stripe_v, out_hbm.at[c, sl])
    if with_deg:
        pltpu.sync_copy(deg_sh.at[sl], stripe_v)
        pltpu.sync_copy(stripe_v, deg_hbm.at[c, sl])


@functools.lru_cache(maxsize=None)
def _make_agg(with_deg):
    part = jax.ShapeDtypeStruct((NC, N_PAD, 16), jnp.float32)
    out_type = (part, part) if with_deg else part
    return pl.kernel(
        functools.partial(_agg_body, with_deg),
        out_type=out_type,
        mesh=plsc.VectorSubcoreMesh(core_axis_name="c", subcore_axis_name="s",
                                    num_cores=NC, num_subcores=NS),
        scratch_types=(
            [
                pltpu.VMEM((NCHUNK, CHUNK), jnp.int32),
                pltpu.VMEM((NCHUNK, CHUNK), jnp.int32),
            ]
            + [pltpu.VMEM((CHUNK, 16), jnp.float32)] * NBUF
            + [
                pltpu.VMEM((ROWS_PER_TILE, 16), jnp.float32),
                pltpu.VMEM((CHUNK, 16), jnp.float32),
                pltpu.VMEM_SHARED((N_PAD, 16), jnp.float32),
                pltpu.VMEM_SHARED((N_PAD, 16), jnp.float32),
            ]
            + [pltpu.SemaphoreType.DMA] * (2 * NBUF)
        ),
        compiler_params=pltpu.CompilerParams(use_tc_tiling_on_sc=False),
    )


def _blockdiag(w):
    tiled = jnp.tile(w, (8, 8))
    r = lax.broadcasted_iota(jnp.int32, (128, 128), 0) // 16
    col = lax.broadcasted_iota(jnp.int32, (128, 128), 1) // 16
    return jnp.where(r == col, tiled, 0.0)


def _node_mid_body(y_ref, p0_ref, p1_ref, d0_ref, d1_ref, w_ref, b_ref, o_ref):
    h = ((p0_ref[...] + p1_ref[...] + y_ref[...])
         / (d0_ref[...] + d1_ref[...] + 1.0) + jnp.tile(b_ref[...], (1, 8)))
    o_ref[...] = jnp.dot(h, _blockdiag(w_ref[...]),
                         preferred_element_type=jnp.float32)


def _node_last_body(y_ref, p0_ref, p1_ref, d0_ref, d1_ref, b_ref, o_ref):
    h = ((p0_ref[...] + p1_ref[...] + y_ref[...])
         / (d0_ref[...] + d1_ref[...] + 1.0) + jnp.tile(b_ref[...], (1, 8)))
    o_ref[...] = jnp.maximum(h, 0.0)


def _node_mid(y, p0, p1, d0, d1, w, b):
    return pl.pallas_call(
        _node_mid_body,
        out_shape=jax.ShapeDtypeStruct((1250, 128), jnp.float32),
    )(y, p0, p1, d0, d1, w, b.reshape(1, 16))


def _node_last(y, p0, p1, d0, d1, b):
    return pl.pallas_call(
        _node_last_body,
        out_shape=jax.ShapeDtypeStruct((1250, 128), jnp.float32),
    )(y, p0, p1, d0, d1, b.reshape(1, 16))


def kernel(in_feat, edge_index, W_ih1, W_hh1, b_ih1, b_hh1,
           W_ih2, W_hh2, b_ih2, b_hh2, W1, b1, W2, b2, W3, b3):
    f32 = jnp.float32
    src = edge_index[0].astype(jnp.int32)
    dst = edge_index[1].astype(jnp.int32)
    pad = E_PAD - N_EDGES
    srcs = jnp.concatenate([src, jnp.zeros((pad,), jnp.int32)])
    dsts = jnp.concatenate([dst, jnp.full((pad,), N_NODES, jnp.int32)])
    srcs = srcs.reshape(NW, NCHUNK, CHUNK)
    dsts = dsts.reshape(NW, NCHUNK, CHUNK)

    xT = jnp.zeros((SEQ, N_LANE_PAD), f32).at[:, :N_NODES].set(in_feat.T)
    y1T = _lstm_project(
        xT,
        _perm_gates(W_ih1, HID1),
        _perm_gates(W_hh1, HID1),
        _perm_gates((b_ih1 + b_hh1).reshape(128, 1), HID1),
        _perm_gates(W_ih2, HID2),
        _perm_gates(W_hh2, HID2),
        _perm_gates((b_ih2 + b_hh2).reshape(64, 1), HID2),
        W1.reshape(SEQ, 16, 16).transpose(0, 2, 1),
    )
    y1 = y1T[:, :N_NODES].T

    a1, dp = _make_agg(True)(y1, srcs, dsts)
    v = lambda p: p[:, :N_NODES, :].reshape(NC, 1250, 128)
    d = v(dp)
    a1 = v(a1)
    y1r = y1.reshape(1250, 128)

    y2r = _node_mid(y1r, a1[0], a1[1], d[0], d[1], W2, b1)
    a2 = v(_make_agg(False)(y2r.reshape(N_NODES, 16), srcs, dsts))
    y3r = _node_mid(y2r, a2[0], a2[1], d[0], d[1], W3, b2)
    a3 = v(_make_agg(False)(y3r.reshape(N_NODES, 16), srcs, dsts))
    outr = _node_last(y3r, a3[0], a3[1], d[0], d[1], b3)
    return outr.reshape(N_NODES, 16).astype(f32)


# speedup vs baseline: 8.7477x; 1.0290x over previous
import functools

import jax
import jax.numpy as jnp
from jax import lax
from jax.experimental import pallas as pl
from jax.experimental.pallas import tpu as pltpu
from jax.experimental.pallas import tpu_sc as plsc

N_NODES = 10000
N_EDGES = 160000
SEQ = 16
HID1 = 32
HID2 = 16

NC = 2
NS = 16
NW = NC * NS
CHUNK = 128
NCHUNK = 40
EPW = CHUNK * NCHUNK
E_PAD = EPW * NW
ROWS_PER_TILE = 632
N_PAD = ROWS_PER_TILE * NS


def _lstm_body(x_ref, wih1_ref, whh1_ref, b1_ref, wih2_ref, whh2_ref,
               b2_ref, w1_ref, out_ref):
    x = x_ref[...]
    wih1 = wih1_ref[...]
    whh1 = whh1_ref[...]
    b1 = b1_ref[...]
    wih2 = wih2_ref[...]
    whh2 = whh2_ref[...]
    b2 = b2_ref[...]
    B = x.shape[1]

    h = jnp.zeros((HID1, B), jnp.float32)
    c = jnp.zeros((HID1, B), jnp.float32)
    h1s = []
    for t in range(SEQ):
        gates = (wih1 * x[t:t + 1, :]
                 + jnp.dot(whh1, h, preferred_element_type=jnp.float32) + b1)
        sio = jax.nn.sigmoid(gates[0:96, :])
        g = jnp.tanh(gates[96:128, :])
        c = sio[32:64, :] * c + sio[0:32, :] * g
        h = sio[64:96, :] * jnp.tanh(c)
        h1s.append(h)

    h1l = jnp.concatenate(h1s, axis=1)
    g2in = jnp.dot(wih2, h1l, preferred_element_type=jnp.float32)

    h2 = jnp.zeros((HID2, B), jnp.float32)
    c2 = jnp.zeros((HID2, B), jnp.float32)
    h2s = []
    for t in range(SEQ):
        gates = (g2in[:, t * B:(t + 1) * B]
                 + jnp.dot(whh2, h2, preferred_element_type=jnp.float32) + b2)
        sio = jax.nn.sigmoid(gates[0:48, :])
        g = jnp.tanh(gates[48:64, :])
        c2 = sio[16:32, :] * c2 + sio[0:16, :] * g
        h2 = sio[32:48, :] * jnp.tanh(c2)
        h2s.append(h2)
    h2stack = jnp.concatenate(h2s, axis=0)
    out_ref[...] = jnp.dot(w1_ref[...], h2stack,
                           preferred_element_type=jnp.float32)


N_LANE_PAD = 10240


def _lstm_project(xT, wih1, whh1, b1, wih2, whh2, b2, w1):
    BN = 2048
    grid = (N_LANE_PAD // BN,)
    full = lambda shape: pl.BlockSpec(shape, lambda i: (0,) * len(shape))
    return pl.pallas_call(
        _lstm_body,
        grid=grid,
        in_specs=[
            pl.BlockSpec((SEQ, BN), lambda i: (0, i)),
            full((128, 1)), full((128, HID1)), full((128, 1)),
            full((64, HID1)), full((64, HID2)), full((64, 1)),
            full((16, SEQ * 16)),
        ],
        out_specs=pl.BlockSpec((16, BN), lambda i: (0, i)),
        out_shape=jax.ShapeDtypeStruct((16, N_LANE_PAD), jnp.float32),
    )(xT, wih1, whh1, b1, wih2, whh2, b2, w1)


def _perm_gates(w, n):
    return w.reshape(4, n, *w.shape[1:])[jnp.array([0, 1, 3, 2])].reshape(w.shape)


NBUF = 8


def _zero_stripe(stripe_v, sh, s):
    def _zrow(i, _):
        stripe_v[i, :] = jnp.zeros((16,), jnp.float32)
        return 0
    lax.fori_loop(0, ROWS_PER_TILE, _zrow, 0)
    pltpu.sync_copy(stripe_v, sh.at[pl.ds(s * ROWS_PER_TILE, ROWS_PER_TILE)])


def _copy_out(sh, stripe_v, out_hbm, c, s):
    sl = pl.ds(s * ROWS_PER_TILE, ROWS_PER_TILE)
    pltpu.sync_copy(sh.at[sl], stripe_v)
    pltpu.sync_copy(stripe_v, out_hbm.at[c, sl])


def _agg_body(y_hbm, srcs_hbm, dsts_hbm, out_hbm, *rest):
    src_v, dst_v = rest[:2]
    rows = rest[2:2 + NBUF]
    stripe_v, acc_sh = rest[2 + NBUF:4 + NBUF]
    sems = rest[4 + NBUF:]
    gsem = sems[:NBUF]
    ssem = sems[NBUF:]

    c = lax.axis_index("c")
    s = lax.axis_index("s")
    wid = s * NC + c

    _zero_stripe(stripe_v, acc_sh, s)
    pltpu.sync_copy(srcs_hbm.at[wid], src_v)
    pltpu.sync_copy(dsts_hbm.at[wid], dst_v)
    plsc.subcore_barrier()

    for b in range(NBUF):
        pltpu.async_copy(y_hbm.at[src_v.at[b]], rows[b], gsem[b])

    def _round(k, _):
        base = k * NBUF
        for b in range(NBUF):
            j = base + b
            pltpu.make_async_copy(y_hbm.at[src_v.at[j]], rows[b],
                                  gsem[b]).wait()
            pltpu.async_copy(rows[b], acc_sh.at[dst_v.at[j]], ssem[b],
                             add=True)

        @pl.when(k < NCHUNK // NBUF - 1)
        def _refill():
            for b in range(NBUF):
                j = base + b
                pltpu.make_async_copy(rows[b], acc_sh.at[dst_v.at[j]],
                                      ssem[b]).wait()
                pltpu.async_copy(y_hbm.at[src_v.at[j + NBUF]], rows[b],
                                 gsem[b])
        return 0
    lax.fori_loop(0, NCHUNK // NBUF, _round, 0)

    for b in range(NBUF):
        j = NCHUNK - NBUF + b
        pltpu.make_async_copy(rows[b], acc_sh.at[dst_v.at[j]], ssem[b]).wait()
    plsc.subcore_barrier()
    _copy_out(acc_sh, stripe_v, out_hbm, c, s)


def _deg_body(dsts_hbm, out_hbm, dst_v, stripe_v, ones_v, deg_sh, sem):
    c = lax.axis_index("c")
    s = lax.axis_index("s")
    wid = s * NC + c

    _zero_stripe(stripe_v, deg_sh, s)

    def _orow(i, _):
        ones_v[i, :] = jnp.ones((16,), jnp.float32)
        return 0
    lax.fori_loop(0, CHUNK, _orow, 0)
    pltpu.sync_copy(dsts_hbm.at[wid], dst_v)
    plsc.subcore_barrier()

    def _fire(j, _):
        pltpu.async_copy(ones_v, deg_sh.at[dst_v.at[j]], sem, add=True)
        return 0
    lax.fori_loop(0, NCHUNK, _fire, 0)

    def _drain(j, _):
        pltpu.make_async_copy(ones_v, deg_sh.at[dst_v.at[0]], sem).wait()
        return 0
    lax.fori_loop(0, NCHUNK, _drain, 0)
    plsc.subcore_barrier()
    _copy_out(deg_sh, stripe_v, out_hbm, c, s)


_PART = jax.ShapeDtypeStruct((NC, N_PAD, 16), jnp.float32)


@functools.lru_cache(maxsize=None)
def _make_agg():
    return pl.kernel(
        _agg_body,
        out_type=_PART,
        mesh=plsc.VectorSubcoreMesh(core_axis_name="c", subcore_axis_name="s",
                                    num_cores=NC, num_subcores=NS),
        scratch_types=(
            [
                pltpu.VMEM((NCHUNK, CHUNK), jnp.int32),
                pltpu.VMEM((NCHUNK, CHUNK), jnp.int32),
            ]
            + [pltpu.VMEM((CHUNK, 16), jnp.float32)] * NBUF
            + [
                pltpu.VMEM((ROWS_PER_TILE, 16), jnp.float32),
                pltpu.VMEM_SHARED((N_PAD, 16), jnp.float32),
            ]
            + [pltpu.SemaphoreType.DMA] * (2 * NBUF)
        ),
        compiler_params=pltpu.CompilerParams(use_tc_tiling_on_sc=False),
    )


@functools.lru_cache(maxsize=None)
def _make_deg():
    return pl.kernel(
        _deg_body,
        out_type=_PART,
        mesh=plsc.VectorSubcoreMesh(core_axis_name="c", subcore_axis_name="s",
                                    num_cores=NC, num_subcores=NS),
        scratch_types=[
            pltpu.VMEM((NCHUNK, CHUNK), jnp.int32),
            pltpu.VMEM((ROWS_PER_TILE, 16), jnp.float32),
            pltpu.VMEM((CHUNK, 16), jnp.float32),
            pltpu.VMEM_SHARED((N_PAD, 16), jnp.float32),
            pltpu.SemaphoreType.DMA,
        ],
        compiler_params=pltpu.CompilerParams(use_tc_tiling_on_sc=False),
    )


def _blockdiag(w):
    tiled = jnp.tile(w, (8, 8))
    r = lax.broadcasted_iota(jnp.int32, (128, 128), 0) // 16
    col = lax.broadcasted_iota(jnp.int32, (128, 128), 1) // 16
    return jnp.where(r == col, tiled, 0.0)


def _node_mid_body(y_ref, p0_ref, p1_ref, d0_ref, d1_ref, w_ref, b_ref, o_ref):
    h = ((p0_ref[...] + p1_ref[...] + y_ref[...])
         / (d0_ref[...] + d1_ref[...] + 1.0) + jnp.tile(b_ref[...], (1, 8)))
    o_ref[...] = jnp.dot(h, _blockdiag(w_ref[...]),
                         preferred_element_type=jnp.float32)


def _node_last_body(y_ref, p0_ref, p1_ref, d0_ref, d1_ref, b_ref, o_ref):
    h = ((p0_ref[...] + p1_ref[...] + y_ref[...])
         / (d0_ref[...] + d1_ref[...] + 1.0) + jnp.tile(b_ref[...], (1, 8)))
    o_ref[...] = jnp.maximum(h, 0.0)


def _node_mid(y, p0, p1, d0, d1, w, b):
    return pl.pallas_call(
        _node_mid_body,
        out_shape=jax.ShapeDtypeStruct((1250, 128), jnp.float32),
    )(y, p0, p1, d0, d1, w, b.reshape(1, 16))


def _node_last(y, p0, p1, d0, d1, b):
    return pl.pallas_call(
        _node_last_body,
        out_shape=jax.ShapeDtypeStruct((1250, 128), jnp.float32),
    )(y, p0, p1, d0, d1, b.reshape(1, 16))


def kernel(in_feat, edge_index, W_ih1, W_hh1, b_ih1, b_hh1,
           W_ih2, W_hh2, b_ih2, b_hh2, W1, b1, W2, b2, W3, b3):
    f32 = jnp.float32
    src = edge_index[0].astype(jnp.int32)
    dst = edge_index[1].astype(jnp.int32)
    pad = E_PAD - N_EDGES
    srcs = jnp.concatenate([src, jnp.zeros((pad,), jnp.int32)])
    dsts = jnp.concatenate([dst, jnp.full((pad,), N_NODES, jnp.int32)])
    srcs = srcs.reshape(NW, NCHUNK, CHUNK)
    dsts = dsts.reshape(NW, NCHUNK, CHUNK)

    xT = jnp.zeros((SEQ, N_LANE_PAD), f32).at[:, :N_NODES].set(in_feat.T)
    y1T = _lstm_project(
        xT,
        _perm_gates(W_ih1, HID1),
        _perm_gates(W_hh1, HID1),
        _perm_gates((b_ih1 + b_hh1).reshape(128, 1), HID1),
        _perm_gates(W_ih2, HID2),
        _perm_gates(W_hh2, HID2),
        _perm_gates((b_ih2 + b_hh2).reshape(64, 1), HID2),
        W1.T,
    )
    y1 = y1T[:, :N_NODES].T

    dp = _make_deg()(dsts)
    a1 = _make_agg()(y1, srcs, dsts)
    v = lambda p: p[:, :N_NODES, :].reshape(NC, 1250, 128)
    d = v(dp)
    a1 = v(a1)
    y1r = y1.reshape(1250, 128)

    y2r = _node_mid(y1r, a1[0], a1[1], d[0], d[1], W2, b1)
    a2 = v(_make_agg()(y2r.reshape(N_NODES, 16), srcs, dsts))
    y3r = _node_mid(y2r, a2[0], a2[1], d[0], d[1], W3, b2)
    a3 = v(_make_agg()(y3r.reshape(N_NODES, 16), srcs, dsts))
    outr = _node_last(y3r, a3[0], a3[1], d[0], d[1], b3)
    return outr.reshape(N_NODES, 16).astype(f32)


# speedup vs baseline: 12.5422x; 1.4338x over previous
import functools

import jax
import jax.numpy as jnp
from jax import lax
from jax.experimental import pallas as pl
from jax.experimental.pallas import tpu as pltpu
from jax.experimental.pallas import tpu_sc as plsc

N_NODES = 10000
N_EDGES = 160000
SEQ = 16
HID1 = 32
HID2 = 16

NC = 2
NS = 16
NW = NC * NS
CHUNK = 128
NCHUNK = 40
EPW = CHUNK * NCHUNK
E_PAD = EPW * NW
ROWS_PER_TILE = 632
N_PAD = ROWS_PER_TILE * NS


def _lstm_body(x_ref, wih1_ref, whh1_ref, b1_ref, wih2_ref, whh2_ref,
               b2_ref, w1_ref, out_ref):
    x = x_ref[...]
    wih1 = wih1_ref[...]
    whh1 = whh1_ref[...]
    b1 = b1_ref[...]
    wih2 = wih2_ref[...]
    whh2 = whh2_ref[...]
    b2 = b2_ref[...]
    B = x.shape[1]

    h = jnp.zeros((HID1, B), jnp.float32)
    c = jnp.zeros((HID1, B), jnp.float32)
    h1s = []
    for t in range(SEQ):
        gates = (wih1 * x[t:t + 1, :]
                 + jnp.dot(whh1, h, preferred_element_type=jnp.float32) + b1)
        sio = jax.nn.sigmoid(gates[0:96, :])
        g = jnp.tanh(gates[96:128, :])
        c = sio[32:64, :] * c + sio[0:32, :] * g
        h = sio[64:96, :] * jnp.tanh(c)
        h1s.append(h)

    h1l = jnp.concatenate(h1s, axis=1)
    g2in = jnp.dot(wih2, h1l, preferred_element_type=jnp.float32)

    h2 = jnp.zeros((HID2, B), jnp.float32)
    c2 = jnp.zeros((HID2, B), jnp.float32)
    h2s = []
    for t in range(SEQ):
        gates = (g2in[:, t * B:(t + 1) * B]
                 + jnp.dot(whh2, h2, preferred_element_type=jnp.float32) + b2)
        sio = jax.nn.sigmoid(gates[0:48, :])
        g = jnp.tanh(gates[48:64, :])
        c2 = sio[16:32, :] * c2 + sio[0:16, :] * g
        h2 = sio[32:48, :] * jnp.tanh(c2)
        h2s.append(h2)
    h2stack = jnp.concatenate(h2s, axis=0)
    out_ref[...] = jnp.dot(w1_ref[...], h2stack,
                           preferred_element_type=jnp.float32)


N_LANE_PAD = 10240


def _lstm_project(xT, wih1, whh1, b1, wih2, whh2, b2, w1):
    BN = 2048
    grid = (N_LANE_PAD // BN,)
    full = lambda shape: pl.BlockSpec(shape, lambda i: (0,) * len(shape))
    return pl.pallas_call(
        _lstm_body,
        grid=grid,
        in_specs=[
            pl.BlockSpec((SEQ, BN), lambda i: (0, i)),
            full((128, 1)), full((128, HID1)), full((128, 1)),
            full((64, HID1)), full((64, HID2)), full((64, 1)),
            full((16, SEQ * 16)),
        ],
        out_specs=pl.BlockSpec((16, BN), lambda i: (0, i)),
        out_shape=jax.ShapeDtypeStruct((16, N_LANE_PAD), jnp.float32),
    )(xT, wih1, whh1, b1, wih2, whh2, b2, w1)


def _perm_gates(w, n):
    return w.reshape(4, n, *w.shape[1:])[jnp.array([0, 1, 3, 2])].reshape(w.shape)


NBUF = 8


def _zero_stripe(stripe_v, sh, s):
    def _zrow(i, _):
        stripe_v[i, :] = jnp.zeros((16,), jnp.float32)
        return 0
    lax.fori_loop(0, ROWS_PER_TILE, _zrow, 0)
    pltpu.sync_copy(stripe_v, sh.at[pl.ds(s * ROWS_PER_TILE, ROWS_PER_TILE)])


def _copy_out(sh, stripe_v, out_hbm, c, s):
    sl = pl.ds(s * ROWS_PER_TILE, ROWS_PER_TILE)
    pltpu.sync_copy(sh.at[sl], stripe_v)
    pltpu.sync_copy(stripe_v, out_hbm.at[c, sl])


def _agg_body(y_hbm, srcs_hbm, dsts_hbm, out_hbm, *rest):
    src_v, dst_v = rest[:2]
    rows = rest[2:2 + NBUF]
    stripe_v, acc_sh = rest[2 + NBUF:4 + NBUF]
    sems = rest[4 + NBUF:]
    gsem = sems[:NBUF]
    ssem = sems[NBUF:]

    c = lax.axis_index("c")
    s = lax.axis_index("s")
    wid = s * NC + c

    _zero_stripe(stripe_v, acc_sh, s)
    pltpu.sync_copy(srcs_hbm.at[wid], src_v)
    pltpu.sync_copy(dsts_hbm.at[wid], dst_v)
    plsc.subcore_barrier()

    for b in range(NBUF):
        pltpu.async_copy(y_hbm.at[src_v.at[b]], rows[b], gsem[b])

    def _round(k, _):
        base = k * NBUF
        for b in range(NBUF):
            j = base + b
            pltpu.make_async_copy(y_hbm.at[src_v.at[j]], rows[b],
                                  gsem[b]).wait()
            pltpu.async_copy(rows[b], acc_sh.at[dst_v.at[j]], ssem[b],
                             add=True)

        @pl.when(k < NCHUNK // NBUF - 1)
        def _refill():
            for b in range(NBUF):
                j = base + b
                pltpu.make_async_copy(rows[b], acc_sh.at[dst_v.at[j]],
                                      ssem[b]).wait()
                pltpu.async_copy(y_hbm.at[src_v.at[j + NBUF]], rows[b],
                                 gsem[b])
        return 0
    lax.fori_loop(0, NCHUNK // NBUF, _round, 0)

    for b in range(NBUF):
        j = NCHUNK - NBUF + b
        pltpu.make_async_copy(rows[b], acc_sh.at[dst_v.at[j]], ssem[b]).wait()
    plsc.subcore_barrier()
    _copy_out(acc_sh, stripe_v, out_hbm, c, s)


def _deg_body(dsts_hbm, out_hbm, dst_v, stripe_v, ones_v, deg_sh, sem):
    c = lax.axis_index("c")
    s = lax.axis_index("s")
    wid = s * NC + c

    _zero_stripe(stripe_v, deg_sh, s)

    def _orow(i, _):
        ones_v[i, :] = jnp.ones((16,), jnp.float32)
        return 0
    lax.fori_loop(0, CHUNK, _orow, 0)
    pltpu.sync_copy(dsts_hbm.at[wid], dst_v)
    plsc.subcore_barrier()

    def _fire(j, _):
        pltpu.async_copy(ones_v, deg_sh.at[dst_v.at[j]], sem, add=True)
        return 0
    lax.fori_loop(0, NCHUNK, _fire, 0)

    def _drain(j, _):
        pltpu.make_async_copy(ones_v, deg_sh.at[dst_v.at[0]], sem).wait()
        return 0
    lax.fori_loop(0, NCHUNK, _drain, 0)
    plsc.subcore_barrier()
    _copy_out(deg_sh, stripe_v, out_hbm, c, s)


_PART = jax.ShapeDtypeStruct((NC, N_PAD, 16), jnp.float32)


@functools.lru_cache(maxsize=None)
def _make_agg():
    return pl.kernel(
        _agg_body,
        out_type=_PART,
        mesh=plsc.VectorSubcoreMesh(core_axis_name="c", subcore_axis_name="s",
                                    num_cores=NC, num_subcores=NS),
        scratch_types=(
            [
                pltpu.VMEM((NCHUNK, CHUNK), jnp.int32),
                pltpu.VMEM((NCHUNK, CHUNK), jnp.int32),
            ]
            + [pltpu.VMEM((CHUNK, 16), jnp.float32)] * NBUF
            + [
                pltpu.VMEM((ROWS_PER_TILE, 16), jnp.float32),
                pltpu.VMEM_SHARED((N_PAD, 16), jnp.float32),
            ]
            + [pltpu.SemaphoreType.DMA] * (2 * NBUF)
        ),
        compiler_params=pltpu.CompilerParams(use_tc_tiling_on_sc=False),
    )


@functools.lru_cache(maxsize=None)
def _make_deg():
    return pl.kernel(
        _deg_body,
        out_type=_PART,
        mesh=plsc.VectorSubcoreMesh(core_axis_name="c", subcore_axis_name="s",
                                    num_cores=NC, num_subcores=NS),
        scratch_types=[
            pltpu.VMEM((NCHUNK, CHUNK), jnp.int32),
            pltpu.VMEM((ROWS_PER_TILE, 16), jnp.float32),
            pltpu.VMEM((CHUNK, 16), jnp.float32),
            pltpu.VMEM_SHARED((N_PAD, 16), jnp.float32),
            pltpu.SemaphoreType.DMA,
        ],
        compiler_params=pltpu.CompilerParams(use_tc_tiling_on_sc=False),
    )


NROW = N_PAD * 16 // 128


def _blockdiag(w):
    tiled = jnp.tile(w, (8, 8))
    r = lax.broadcasted_iota(jnp.int32, (128, 128), 0) // 16
    col = lax.broadcasted_iota(jnp.int32, (128, 128), 1) // 16
    return jnp.where(r == col, tiled, 0.0)


def _node_h(y_ref, p_ref, d_ref, b_ref):
    return ((p_ref[0, :, :] + p_ref[1, :, :] + y_ref[...])
            / (d_ref[0, :, :] + d_ref[1, :, :] + 1.0)
            + jnp.tile(b_ref[...], (1, 8)))


def _node_mid_body(y_ref, p_ref, d_ref, w_ref, b_ref, o_ref):
    o_ref[...] = jnp.dot(_node_h(y_ref, p_ref, d_ref, b_ref),
                         _blockdiag(w_ref[...]),
                         preferred_element_type=jnp.float32)


def _node_last_body(y_ref, p_ref, d_ref, b_ref, o_ref):
    o_ref[...] = jnp.maximum(_node_h(y_ref, p_ref, d_ref, b_ref), 0.0)


def _node_mid(y, p, d, w, b):
    return pl.pallas_call(
        _node_mid_body,
        out_shape=jax.ShapeDtypeStruct((NROW, 128), jnp.float32),
    )(y, p, d, w, b.reshape(1, 16))


def _node_last(y, p, d, b):
    return pl.pallas_call(
        _node_last_body,
        out_shape=jax.ShapeDtypeStruct((NROW, 128), jnp.float32),
    )(y, p, d, b.reshape(1, 16))


def kernel(in_feat, edge_index, W_ih1, W_hh1, b_ih1, b_hh1,
           W_ih2, W_hh2, b_ih2, b_hh2, W1, b1, W2, b2, W3, b3):
    f32 = jnp.float32
    src = edge_index[0].astype(jnp.int32)
    dst = edge_index[1].astype(jnp.int32)
    pad = E_PAD - N_EDGES
    srcs = jnp.concatenate([src, jnp.zeros((pad,), jnp.int32)])
    dsts = jnp.concatenate(
        [dst, N_NODES + (jnp.arange(pad, dtype=jnp.int32) % (N_PAD - N_NODES))])
    srcs = srcs.reshape(NW, NCHUNK, CHUNK)
    dsts = dsts.reshape(NW, NCHUNK, CHUNK)

    xT = jnp.zeros((SEQ, N_LANE_PAD), f32).at[:, :N_NODES].set(in_feat.T)
    y1T = _lstm_project(
        xT,
        _perm_gates(W_ih1, HID1),
        _perm_gates(W_hh1, HID1),
        _perm_gates((b_ih1 + b_hh1).reshape(128, 1), HID1),
        _perm_gates(W_ih2, HID2),
        _perm_gates(W_hh2, HID2),
        _perm_gates((b_ih2 + b_hh2).reshape(64, 1), HID2),
        W1.T,
    )

    y1 = y1T[:, :N_PAD].T
    dp = _make_deg()(dsts)
    a1 = _make_agg()(y1, srcs, dsts)
    v = lambda p: p.reshape(NC, NROW, 128)
    d = v(dp)

    y2r = _node_mid(y1.reshape(NROW, 128), v(a1), d, W2, b1)
    a2 = _make_agg()(y2r.reshape(N_PAD, 16), srcs, dsts)
    y3r = _node_mid(y2r, v(a2), d, W3, b2)
    a3 = _make_agg()(y3r.reshape(N_PAD, 16), srcs, dsts)
    outr = _node_last(y3r, v(a3), d, b3)
    return outr.reshape(N_PAD, 16)[:N_NODES].astype(f32)


# speedup vs baseline: 16.6865x; 1.3304x over previous
import functools

import jax
import jax.numpy as jnp
from jax import lax
from jax.experimental import pallas as pl
from jax.experimental.pallas import tpu as pltpu
from jax.experimental.pallas import tpu_sc as plsc

N_NODES = 10000
N_EDGES = 160000
SEQ = 16
HID1 = 32
HID2 = 16

NC = 2
NS = 16
NW = NC * NS
CHUNK = 128
NCHUNK = 40
EPW = CHUNK * NCHUNK
E_PAD = EPW * NW
ROWS_PER_TILE = 640
N_PAD = ROWS_PER_TILE * NS


def _lstm_body(x_ref, wih1_ref, whh1_ref, b1_ref, wih2_ref, whh2_ref,
               b2_ref, w1_ref, out_ref):
    x = x_ref[...]
    wih1 = wih1_ref[...]
    whh1 = whh1_ref[...]
    b1 = b1_ref[...]
    wih2 = wih2_ref[...]
    whh2 = whh2_ref[...]
    b2 = b2_ref[...]
    B = x.shape[1]

    h = jnp.zeros((HID1, B), jnp.float32)
    c = jnp.zeros((HID1, B), jnp.float32)
    h1s = []
    for t in range(SEQ):
        gates = (wih1 * x[t:t + 1, :]
                 + jnp.dot(whh1, h, preferred_element_type=jnp.float32) + b1)
        sio = jax.nn.sigmoid(gates[0:96, :])
        g = jnp.tanh(gates[96:128, :])
        c = sio[32:64, :] * c + sio[0:32, :] * g
        h = sio[64:96, :] * jnp.tanh(c)
        h1s.append(h)

    h1l = jnp.concatenate(h1s, axis=1)
    g2in = jnp.dot(wih2, h1l, preferred_element_type=jnp.float32)

    h2 = jnp.zeros((HID2, B), jnp.float32)
    c2 = jnp.zeros((HID2, B), jnp.float32)
    h2s = []
    for t in range(SEQ):
        gates = (g2in[:, t * B:(t + 1) * B]
                 + jnp.dot(whh2, h2, preferred_element_type=jnp.float32) + b2)
        sio = jax.nn.sigmoid(gates[0:48, :])
        g = jnp.tanh(gates[48:64, :])
        c2 = sio[16:32, :] * c2 + sio[0:16, :] * g
        h2 = sio[32:48, :] * jnp.tanh(c2)
        h2s.append(h2)
    h2stack = jnp.concatenate(h2s, axis=0)
    acc = jnp.dot(w1_ref[...], h2stack,
                  preferred_element_type=jnp.float32)
    out_ref[...] = acc.T


def _lstm_project(xT, wih1, whh1, b1, wih2, whh2, b2, w1):
    BN = 2048
    grid = (N_PAD // BN,)
    full = lambda shape: pl.BlockSpec(shape, lambda i: (0,) * len(shape))
    return pl.pallas_call(
        _lstm_body,
        grid=grid,
        in_specs=[
            pl.BlockSpec((SEQ, BN), lambda i: (0, i)),
            full((128, 1)), full((128, HID1)), full((128, 1)),
            full((64, HID1)), full((64, HID2)), full((64, 1)),
            full((16, SEQ * 16)),
        ],
        out_specs=pl.BlockSpec((BN, 16), lambda i: (i, 0)),
        out_shape=jax.ShapeDtypeStruct((N_PAD, 16), jnp.float32),
    )(xT, wih1, whh1, b1, wih2, whh2, b2, w1)


def _perm_gates(w, n):
    return w.reshape(4, n, *w.shape[1:])[jnp.array([0, 1, 3, 2])].reshape(w.shape)


NBUF = 8


def _zero_stripe(stripe_v, sh, s):
    def _zrow(i, _):
        stripe_v[i, :] = jnp.zeros((16,), jnp.float32)
        return 0
    lax.fori_loop(0, ROWS_PER_TILE, _zrow, 0)
    pltpu.sync_copy(stripe_v, sh.at[pl.ds(s * ROWS_PER_TILE, ROWS_PER_TILE)])


def _copy_out(sh, stripe_v, out_hbm, c, s):
    sl = pl.ds(s * ROWS_PER_TILE, ROWS_PER_TILE)
    pltpu.sync_copy(sh.at[sl], stripe_v)
    pltpu.sync_copy(stripe_v, out_hbm.at[c, sl])


def _agg_body(y_hbm, srcs_hbm, dsts_hbm, out_hbm, *rest):
    src_v, dst_v = rest[:2]
    rows = rest[2:2 + NBUF]
    stripe_v, y_sh, acc_sh = rest[2 + NBUF:5 + NBUF]
    sems = rest[5 + NBUF:]
    gsem = sems[:NBUF]
    ssem = sems[NBUF:]

    c = lax.axis_index("c")
    s = lax.axis_index("s")
    wid = s * NC + c

    sl = pl.ds(s * ROWS_PER_TILE, ROWS_PER_TILE)
    pltpu.sync_copy(y_hbm.at[sl], stripe_v)
    pltpu.sync_copy(stripe_v, y_sh.at[sl])
    _zero_stripe(stripe_v, acc_sh, s)
    pltpu.sync_copy(srcs_hbm.at[wid], src_v)
    pltpu.sync_copy(dsts_hbm.at[wid], dst_v)
    plsc.subcore_barrier()

    for b in range(NBUF):
        pltpu.async_copy(y_sh.at[src_v.at[b]], rows[b], gsem[b])

    def _round(k, _):
        base = k * NBUF
        for b in range(NBUF):
            j = base + b
            pltpu.make_async_copy(y_sh.at[src_v.at[j]], rows[b],
                                  gsem[b]).wait()
            pltpu.async_copy(rows[b], acc_sh.at[dst_v.at[j]], ssem[b],
                             add=True)

        @pl.when(k < NCHUNK // NBUF - 1)
        def _refill():
            for b in range(NBUF):
                j = base + b
                pltpu.make_async_copy(rows[b], acc_sh.at[dst_v.at[j]],
                                      ssem[b]).wait()
                pltpu.async_copy(y_sh.at[src_v.at[j + NBUF]], rows[b],
                                 gsem[b])
        return 0
    lax.fori_loop(0, NCHUNK // NBUF, _round, 0)

    for b in range(NBUF):
        j = NCHUNK - NBUF + b
        pltpu.make_async_copy(rows[b], acc_sh.at[dst_v.at[j]], ssem[b]).wait()
    plsc.subcore_barrier()
    _copy_out(acc_sh, stripe_v, out_hbm, c, s)


def _deg_body(dsts_hbm, out_hbm, dst_v, stripe_v, ones_v, deg_sh, sem):
    c = lax.axis_index("c")
    s = lax.axis_index("s")
    wid = s * NC + c

    _zero_stripe(stripe_v, deg_sh, s)

    def _orow(i, _):
        ones_v[i, :] = jnp.ones((16,), jnp.float32)
        return 0
    lax.fori_loop(0, CHUNK, _orow, 0)
    pltpu.sync_copy(dsts_hbm.at[wid], dst_v)
    plsc.subcore_barrier()

    def _fire(j, _):
        pltpu.async_copy(ones_v, deg_sh.at[dst_v.at[j]], sem, add=True)
        return 0
    lax.fori_loop(0, NCHUNK, _fire, 0)

    def _drain(j, _):
        pltpu.make_async_copy(ones_v, deg_sh.at[dst_v.at[0]], sem).wait()
        return 0
    lax.fori_loop(0, NCHUNK, _drain, 0)
    plsc.subcore_barrier()
    _copy_out(deg_sh, stripe_v, out_hbm, c, s)


_PART = jax.ShapeDtypeStruct((NC, N_PAD, 16), jnp.float32)


@functools.lru_cache(maxsize=None)
def _make_agg():
    return pl.kernel(
        _agg_body,
        out_type=_PART,
        mesh=plsc.VectorSubcoreMesh(core_axis_name="c", subcore_axis_name="s",
                                    num_cores=NC, num_subcores=NS),
        scratch_types=(
            [
                pltpu.VMEM((NCHUNK, CHUNK), jnp.int32),
                pltpu.VMEM((NCHUNK, CHUNK), jnp.int32),
            ]
            + [pltpu.VMEM((CHUNK, 16), jnp.float32)] * NBUF
            + [
                pltpu.VMEM((ROWS_PER_TILE, 16), jnp.float32),
                pltpu.VMEM_SHARED((N_PAD, 16), jnp.float32),
                pltpu.VMEM_SHARED((N_PAD, 16), jnp.float32),
            ]
            + [pltpu.SemaphoreType.DMA] * (2 * NBUF)
        ),
        compiler_params=pltpu.CompilerParams(use_tc_tiling_on_sc=False),
    )


@functools.lru_cache(maxsize=None)
def _make_deg():
    return pl.kernel(
        _deg_body,
        out_type=_PART,
        mesh=plsc.VectorSubcoreMesh(core_axis_name="c", subcore_axis_name="s",
                                    num_cores=NC, num_subcores=NS),
        scratch_types=[
            pltpu.VMEM((NCHUNK, CHUNK), jnp.int32),
            pltpu.VMEM((ROWS_PER_TILE, 16), jnp.float32),
            pltpu.VMEM((CHUNK, 16), jnp.float32),
            pltpu.VMEM_SHARED((N_PAD, 16), jnp.float32),
            pltpu.SemaphoreType.DMA,
        ],
        compiler_params=pltpu.CompilerParams(use_tc_tiling_on_sc=False),
    )


NROW = N_PAD * 16 // 128


def _blockdiag(w):
    tiled = jnp.tile(w, (8, 8))
    r = lax.broadcasted_iota(jnp.int32, (128, 128), 0) // 16
    col = lax.broadcasted_iota(jnp.int32, (128, 128), 1) // 16
    return jnp.where(r == col, tiled, 0.0)


def _node_h(y_ref, p_ref, d_ref, b_ref):
    return ((p_ref[0, :, :] + p_ref[1, :, :] + y_ref[...])
            / (d_ref[0, :, :] + d_ref[1, :, :] + 1.0)
            + jnp.tile(b_ref[...], (1, 8)))


def _node_mid_body(y_ref, p_ref, d_ref, w_ref, b_ref, o_ref):
    o_ref[...] = jnp.dot(_node_h(y_ref, p_ref, d_ref, b_ref),
                         _blockdiag(w_ref[...]),
                         preferred_element_type=jnp.float32)


def _node_last_body(y_ref, p_ref, d_ref, b_ref, o_ref):
    o_ref[...] = jnp.maximum(_node_h(y_ref, p_ref, d_ref, b_ref), 0.0)


def _node_mid(y, p, d, w, b):
    return pl.pallas_call(
        _node_mid_body,
        out_shape=jax.ShapeDtypeStruct((NROW, 128), jnp.float32),
    )(y, p, d, w, b.reshape(1, 16))


def _node_last(y, p, d, b):
    return pl.pallas_call(
        _node_last_body,
        out_shape=jax.ShapeDtypeStruct((NROW, 128), jnp.float32),
    )(y, p, d, b.reshape(1, 16))


def kernel(in_feat, edge_index, W_ih1, W_hh1, b_ih1, b_hh1,
           W_ih2, W_hh2, b_ih2, b_hh2, W1, b1, W2, b2, W3, b3):
    f32 = jnp.float32
    src = edge_index[0].astype(jnp.int32)
    dst = edge_index[1].astype(jnp.int32)
    pad = E_PAD - N_EDGES
    srcs = jnp.concatenate([src, jnp.zeros((pad,), jnp.int32)])
    dsts = jnp.concatenate(
        [dst, N_NODES + (jnp.arange(pad, dtype=jnp.int32) % (N_PAD - N_NODES))])
    srcs = srcs.reshape(NW, NCHUNK, CHUNK)
    dsts = dsts.reshape(NW, NCHUNK, CHUNK)

    xT = jnp.zeros((SEQ, N_PAD), f32).at[:, :N_NODES].set(in_feat.T)
    y1 = _lstm_project(
        xT,
        _perm_gates(W_ih1, HID1),
        _perm_gates(W_hh1, HID1),
        _perm_gates((b_ih1 + b_hh1).reshape(128, 1), HID1),
        _perm_gates(W_ih2, HID2),
        _perm_gates(W_hh2, HID2),
        _perm_gates((b_ih2 + b_hh2).reshape(64, 1), HID2),
        W1.T,
    )

    dp = _make_deg()(dsts)
    a1 = _make_agg()(y1, srcs, dsts)
    v = lambda p: p.reshape(NC, NROW, 128)
    d = v(dp)

    y2r = _node_mid(y1.reshape(NROW, 128), v(a1), d, W2, b1)
    a2 = _make_agg()(y2r.reshape(N_PAD, 16), srcs, dsts)
    y3r = _node_mid(y2r, v(a2), d, W3, b2)
    a3 = _make_agg()(y3r.reshape(N_PAD, 16), srcs, dsts)
    outr = _node_last(y3r, v(a3), d, b3)
    return outr.reshape(N_PAD, 16)[:N_NODES].astype(f32)


# speedup vs baseline: 17.3120x; 1.0375x over previous
import functools

import jax
import jax.numpy as jnp
from jax import lax
from jax.experimental import pallas as pl
from jax.experimental.pallas import tpu as pltpu
from jax.experimental.pallas import tpu_sc as plsc

N_NODES = 10000
N_EDGES = 160000
SEQ = 16
HID1 = 32
HID2 = 16

NC = 2
NS = 16
NW = NC * NS
CHUNK = 128
NCHUNK = 40
EPW = CHUNK * NCHUNK
E_PAD = EPW * NW
ROWS_PER_TILE = 640
N_PAD = ROWS_PER_TILE * NS


def _lstm_body(x_ref, wih1_ref, whh1_ref, b1_ref, wih2_ref, whh2_ref,
               b2_ref, w1_ref, dep_ref, out_ref):
    del dep_ref
    x = x_ref[...]
    wih1 = wih1_ref[...]
    whh1 = whh1_ref[...]
    b1 = b1_ref[...]
    wih2 = wih2_ref[...]
    whh2 = whh2_ref[...]
    b2 = b2_ref[...]
    B = x.shape[1]

    h = jnp.zeros((HID1, B), jnp.float32)
    c = jnp.zeros((HID1, B), jnp.float32)
    h1s = []
    for t in range(SEQ):
        gates = (wih1 * x[t:t + 1, :]
                 + jnp.dot(whh1, h, preferred_element_type=jnp.float32) + b1)
        sio = jax.nn.sigmoid(gates[0:96, :])
        g = jnp.tanh(gates[96:128, :])
        c = sio[32:64, :] * c + sio[0:32, :] * g
        h = sio[64:96, :] * jnp.tanh(c)
        h1s.append(h)

    h1l = jnp.concatenate(h1s, axis=1)
    g2in = jnp.dot(wih2, h1l, preferred_element_type=jnp.float32)

    h2 = jnp.zeros((HID2, B), jnp.float32)
    c2 = jnp.zeros((HID2, B), jnp.float32)
    h2s = []
    for t in range(SEQ):
        gates = (g2in[:, t * B:(t + 1) * B]
                 + jnp.dot(whh2, h2, preferred_element_type=jnp.float32) + b2)
        sio = jax.nn.sigmoid(gates[0:48, :])
        g = jnp.tanh(gates[48:64, :])
        c2 = sio[16:32, :] * c2 + sio[0:16, :] * g
        h2 = sio[32:48, :] * jnp.tanh(c2)
        h2s.append(h2)
    h2stack = jnp.concatenate(h2s, axis=0)
    acc = jnp.dot(w1_ref[...], h2stack,
                  preferred_element_type=jnp.float32)
    out_ref[...] = acc.T


def _lstm_project(xT, wih1, whh1, b1, wih2, whh2, b2, w1, dep):
    BN = 2048
    grid = (N_PAD // BN,)
    full = lambda shape: pl.BlockSpec(shape, lambda i: (0,) * len(shape))
    return pl.pallas_call(
        _lstm_body,
        grid=grid,
        in_specs=[
            pl.BlockSpec((SEQ, BN), lambda i: (0, i)),
            full((128, 1)), full((128, HID1)), full((128, 1)),
            full((64, HID1)), full((64, HID2)), full((64, 1)),
            full((16, SEQ * 16)),
            pl.BlockSpec((1, 1, 16), lambda i: (0, 0, 0)),
        ],
        out_specs=pl.BlockSpec((BN, 16), lambda i: (i, 0)),
        out_shape=jax.ShapeDtypeStruct((N_PAD, 16), jnp.float32),
    )(xT, wih1, whh1, b1, wih2, whh2, b2, w1, dep[:1, :1, :])


def _perm_gates(w, n):
    return w.reshape(4, n, *w.shape[1:])[jnp.array([0, 1, 3, 2])].reshape(w.shape)


NBUF = 8


def _zero_stripe(stripe_v, sh, s):
    def _zrow(i, _):
        stripe_v[i, :] = jnp.zeros((16,), jnp.float32)
        return 0
    lax.fori_loop(0, ROWS_PER_TILE, _zrow, 0)
    pltpu.sync_copy(stripe_v, sh.at[pl.ds(s * ROWS_PER_TILE, ROWS_PER_TILE)])


def _copy_out(sh, stripe_v, out_hbm, c, s):
    sl = pl.ds(s * ROWS_PER_TILE, ROWS_PER_TILE)
    pltpu.sync_copy(sh.at[sl], stripe_v)
    pltpu.sync_copy(stripe_v, out_hbm.at[c, sl])


def _agg_body(y_hbm, srcs_hbm, dsts_hbm, out_hbm, *rest):
    src_v, dst_v = rest[:2]
    rows = rest[2:2 + NBUF]
    stripe_v, y_sh, acc_sh = rest[2 + NBUF:5 + NBUF]
    sems = rest[5 + NBUF:]
    gsem = sems[:NBUF]
    ssem = sems[NBUF:]

    c = lax.axis_index("c")
    s = lax.axis_index("s")
    wid = s * NC + c

    sl = pl.ds(s * ROWS_PER_TILE, ROWS_PER_TILE)
    pltpu.sync_copy(y_hbm.at[sl], stripe_v)
    pltpu.sync_copy(stripe_v, y_sh.at[sl])
    _zero_stripe(stripe_v, acc_sh, s)
    pltpu.sync_copy(srcs_hbm.at[wid], src_v)
    pltpu.sync_copy(dsts_hbm.at[wid], dst_v)
    plsc.subcore_barrier()

    for b in range(NBUF):
        pltpu.async_copy(y_sh.at[src_v.at[b]], rows[b], gsem[b])

    def _round(k, _):
        base = k * NBUF
        for b in range(NBUF):
            j = base + b
            pltpu.make_async_copy(y_sh.at[src_v.at[j]], rows[b],
                                  gsem[b]).wait()
            pltpu.async_copy(rows[b], acc_sh.at[dst_v.at[j]], ssem[b],
                             add=True)

        @pl.when(k < NCHUNK // NBUF - 1)
        def _refill():
            for b in range(NBUF):
                j = base + b
                pltpu.make_async_copy(rows[b], acc_sh.at[dst_v.at[j]],
                                      ssem[b]).wait()
                pltpu.async_copy(y_sh.at[src_v.at[j + NBUF]], rows[b],
                                 gsem[b])
        return 0
    lax.fori_loop(0, NCHUNK // NBUF, _round, 0)

    for b in range(NBUF):
        j = NCHUNK - NBUF + b
        pltpu.make_async_copy(rows[b], acc_sh.at[dst_v.at[j]], ssem[b]).wait()
    plsc.subcore_barrier()
    _copy_out(acc_sh, stripe_v, out_hbm, c, s)


def _deg_body(dsts_hbm, out_hbm, dst_v, stripe_v, ones_v, deg_sh, sem):
    c = lax.axis_index("c")
    s = lax.axis_index("s")
    wid = s * NC + c

    _zero_stripe(stripe_v, deg_sh, s)

    def _orow(i, _):
        ones_v[i, :] = jnp.ones((16,), jnp.float32)
        return 0
    lax.fori_loop(0, CHUNK, _orow, 0)
    pltpu.sync_copy(dsts_hbm.at[wid], dst_v)
    plsc.subcore_barrier()

    def _fire(j, _):
        pltpu.async_copy(ones_v, deg_sh.at[dst_v.at[j]], sem, add=True)
        return 0
    lax.fori_loop(0, NCHUNK, _fire, 0)

    def _drain(j, _):
        pltpu.make_async_copy(ones_v, deg_sh.at[dst_v.at[0]], sem).wait()
        return 0
    lax.fori_loop(0, NCHUNK, _drain, 0)
    plsc.subcore_barrier()
    _copy_out(deg_sh, stripe_v, out_hbm, c, s)


_PART = jax.ShapeDtypeStruct((NC, N_PAD, 16), jnp.float32)


@functools.lru_cache(maxsize=None)
def _make_agg():
    return pl.kernel(
        _agg_body,
        out_type=_PART,
        mesh=plsc.VectorSubcoreMesh(core_axis_name="c", subcore_axis_name="s",
                                    num_cores=NC, num_subcores=NS),
        scratch_types=(
            [
                pltpu.VMEM((NCHUNK, CHUNK), jnp.int32),
                pltpu.VMEM((NCHUNK, CHUNK), jnp.int32),
            ]
            + [pltpu.VMEM((CHUNK, 16), jnp.float32)] * NBUF
            + [
                pltpu.VMEM((ROWS_PER_TILE, 16), jnp.float32),
                pltpu.VMEM_SHARED((N_PAD, 16), jnp.float32),
                pltpu.VMEM_SHARED((N_PAD, 16), jnp.float32),
            ]
            + [pltpu.SemaphoreType.DMA] * (2 * NBUF)
        ),
        compiler_params=pltpu.CompilerParams(use_tc_tiling_on_sc=False),
    )


@functools.lru_cache(maxsize=None)
def _make_deg():
    return pl.kernel(
        _deg_body,
        out_type=_PART,
        mesh=plsc.VectorSubcoreMesh(core_axis_name="c", subcore_axis_name="s",
                                    num_cores=NC, num_subcores=NS),
        scratch_types=[
            pltpu.VMEM((NCHUNK, CHUNK), jnp.int32),
            pltpu.VMEM((ROWS_PER_TILE, 16), jnp.float32),
            pltpu.VMEM((CHUNK, 16), jnp.float32),
            pltpu.VMEM_SHARED((N_PAD, 16), jnp.float32),
            pltpu.SemaphoreType.DMA,
        ],
        compiler_params=pltpu.CompilerParams(use_tc_tiling_on_sc=False),
    )


NROW = N_PAD * 16 // 128


def _blockdiag(w):
    tiled = jnp.tile(w, (8, 8))
    r = lax.broadcasted_iota(jnp.int32, (128, 128), 0) // 16
    col = lax.broadcasted_iota(jnp.int32, (128, 128), 1) // 16
    return jnp.where(r == col, tiled, 0.0)


def _node_h(y_ref, p_ref, d_ref, b_ref):
    return ((p_ref[0, :, :] + p_ref[1, :, :] + y_ref[...])
            / (d_ref[0, :, :] + d_ref[1, :, :] + 1.0)
            + jnp.tile(b_ref[...], (1, 8)))


def _node_mid_body(y_ref, p_ref, d_ref, w_ref, b_ref, o_ref):
    o_ref[...] = jnp.dot(_node_h(y_ref, p_ref, d_ref, b_ref),
                         _blockdiag(w_ref[...]),
                         preferred_element_type=jnp.float32)


def _node_last_body(y_ref, p_ref, d_ref, b_ref, o_ref):
    o_ref[...] = jnp.maximum(_node_h(y_ref, p_ref, d_ref, b_ref), 0.0)


def _node_mid(y, p, d, w, b):
    return pl.pallas_call(
        _node_mid_body,
        out_shape=jax.ShapeDtypeStruct((NROW, 128), jnp.float32),
    )(y, p, d, w, b.reshape(1, 16))


def _node_last(y, p, d, b):
    return pl.pallas_call(
        _node_last_body,
        out_shape=jax.ShapeDtypeStruct((NROW, 128), jnp.float32),
    )(y, p, d, b.reshape(1, 16))


def kernel(in_feat, edge_index, W_ih1, W_hh1, b_ih1, b_hh1,
           W_ih2, W_hh2, b_ih2, b_hh2, W1, b1, W2, b2, W3, b3):
    f32 = jnp.float32
    src = edge_index[0].astype(jnp.int32)
    dst = edge_index[1].astype(jnp.int32)
    pad = E_PAD - N_EDGES
    srcs = jnp.concatenate([src, jnp.zeros((pad,), jnp.int32)])
    dsts = jnp.concatenate(
        [dst, N_NODES + (jnp.arange(pad, dtype=jnp.int32) % (N_PAD - N_NODES))])
    srcs = srcs.reshape(NW, NCHUNK, CHUNK)
    dsts = dsts.reshape(NW, NCHUNK, CHUNK)

    dp = _make_deg()(dsts)
    xT = jnp.zeros((SEQ, N_PAD), f32).at[:, :N_NODES].set(in_feat.T)
    y1 = _lstm_project(
        xT,
        _perm_gates(W_ih1, HID1),
        _perm_gates(W_hh1, HID1),
        _perm_gates((b_ih1 + b_hh1).reshape(128, 1), HID1),
        _perm_gates(W_ih2, HID2),
        _perm_gates(W_hh2, HID2),
        _perm_gates((b_ih2 + b_hh2).reshape(64, 1), HID2),
        W1.T,
        dp,
    )

    a1 = _make_agg()(y1, srcs, dsts)
    v = lambda p: p.reshape(NC, NROW, 128)
    d = v(dp)

    y2r = _node_mid(y1.reshape(NROW, 128), v(a1), d, W2, b1)
    a2 = _make_agg()(y2r.reshape(N_PAD, 16), srcs, dsts)
    y3r = _node_mid(y2r, v(a2), d, W3, b2)
    a3 = _make_agg()(y3r.reshape(N_PAD, 16), srcs, dsts)
    outr = _node_last(y3r, v(a3), d, b3)
    return outr.reshape(N_PAD, 16)[:N_NODES].astype(f32)
